# baseline (speedup 1.0000x reference)
import functools

import jax
import jax.numpy as jnp
from jax import lax
from jax.experimental import pallas as pl
from jax.experimental.pallas import tpu as pltpu
from jax.experimental.pallas import tpu_sc as plsc

N_VOCAB = 100000
D_MODEL = 128
BATCH = 4
SEQ = 8192

NUM_CORES = 2
NUM_SUBCORES = 16
NUM_WORKERS = NUM_CORES * NUM_SUBCORES
S_PER_W = SEQ // NUM_WORKERS
SUB = 128
SPB = S_PER_W // SUB
K = BATCH * SPB
LANES = 16
RING = 5
PRIME = 4

_mesh = plsc.VectorSubcoreMesh(core_axis_name="c", subcore_axis_name="s")


@functools.partial(
    pl.kernel,
    mesh=_mesh,
    out_type=jax.ShapeDtypeStruct((BATCH, SEQ, D_MODEL), jnp.float32),
    scratch_types=[
        pltpu.VMEM((K, SUB), jnp.int32),
        pltpu.VMEM((S_PER_W, D_MODEL), jnp.float32),
    ] + [pltpu.VMEM((SUB, D_MODEL), jnp.float32) for _ in range(RING)]
      + [pltpu.SemaphoreType.DMA for _ in range(2 * RING + 2)],
)
def _emb_kernel(x_hbm, emb_hbm, pos_hbm, out_hbm, idx_v, pos_v, *rest):
    bufs = rest[:RING]
    sems = rest[RING:]
    gsems = sems[:RING]
    osems = sems[RING:2 * RING]
    psem = sems[2 * RING]
    isem = sems[2 * RING + 1]

    wid = lax.axis_index("s") * NUM_CORES + lax.axis_index("c")
    s0 = wid * S_PER_W

    def stage_idx(k, sem):
        return pltpu.async_copy(
            x_hbm.at[k // SPB, pl.ds(s0 + (k % SPB) * SUB, SUB)],
            idx_v.at[k], sem)

    early = [stage_idx(k, osems[k]) for k in range(PRIME)]
    late = [stage_idx(k, isem) for k in range(PRIME, K)]

    pos_copy = pltpu.async_copy(pos_hbm.at[pl.ds(s0, S_PER_W)], pos_v, psem)

    def gather_block(k):
        return pltpu.async_copy(
            emb_hbm.at[idx_v.at[k]], bufs[k % RING], gsems[k % RING])

    gathers = [None] * RING
    outs = [None] * RING
    for k in range(PRIME):
        early[k].wait()
        gathers[k] = gather_block(k)

    for c in late:
        c.wait()
    pos_copy.wait()

    for k in range(K):
        sl = k % RING
        kp = k + PRIME
        if kp < K:
            ps = kp % RING
            if outs[ps] is not None:
                outs[ps].wait()
                outs[ps] = None
            gathers[ps] = gather_block(kp)

        cur = bufs[sl]
        gathers[sl].wait()

        def add_pos(r, carry, cur=cur, base=(k % SPB) * SUB):
            for cc in range(D_MODEL // LANES):
                c = cc * LANES
                cur[r, pl.ds(c, LANES)] = (
                    cur[r, pl.ds(c, LANES)] + pos_v[base + r, pl.ds(c, LANES)])
            return carry

        lax.fori_loop(0, SUB, add_pos, 0)

        outs[sl] = pltpu.async_copy(
            cur, out_hbm.at[k // SPB, pl.ds(s0 + (k % SPB) * SUB, SUB)],
            osems[sl])

    for sl in range(RING):
        if outs[sl] is not None:
            outs[sl].wait()


def kernel(x, emb, pos_emb):
    return _emb_kernel(x.astype(jnp.int32), emb, pos_emb)


# speedup vs baseline: 1.0620x; 1.0620x over previous
import functools

import jax
import jax.numpy as jnp
from jax import lax
from jax.experimental import pallas as pl
from jax.experimental.pallas import tpu as pltpu
from jax.experimental.pallas import tpu_sc as plsc

N_VOCAB = 100000
D_MODEL = 128
BATCH = 4
SEQ = 8192

NUM_CORES = 2
NUM_SUBCORES = 16
NUM_WORKERS = NUM_CORES * NUM_SUBCORES
S_PER_W = SEQ // NUM_WORKERS
SUB = 128
SPB = S_PER_W // SUB
K = BATCH * SPB
LANES = 16
RING = 5
PRIME = 3

_mesh = plsc.VectorSubcoreMesh(core_axis_name="c", subcore_axis_name="s")


@functools.partial(
    pl.kernel,
    mesh=_mesh,
    out_type=jax.ShapeDtypeStruct((BATCH, SEQ, D_MODEL), jnp.float32),
    scratch_types=[
        pltpu.VMEM((K, SUB), jnp.int32),
        pltpu.VMEM((S_PER_W, D_MODEL), jnp.float32),
    ] + [pltpu.VMEM((SUB, D_MODEL), jnp.float32) for _ in range(RING)]
      + [pltpu.SemaphoreType.DMA for _ in range(2 * RING + 2)],
)
def _emb_kernel(x_hbm, emb_hbm, pos_hbm, out_hbm, idx_v, pos_v, *rest):
    bufs = rest[:RING]
    sems = rest[RING:]
    gsems = sems[:RING]
    osems = sems[RING:2 * RING]
    psem = sems[2 * RING]
    isem = sems[2 * RING + 1]

    wid = lax.axis_index("s") * NUM_CORES + lax.axis_index("c")
    s0 = wid * S_PER_W

    def stage_idx(k, sem):
        return pltpu.async_copy(
            x_hbm.at[k // SPB, pl.ds(s0 + (k % SPB) * SUB, SUB)],
            idx_v.at[k], sem)

    early = [stage_idx(k, osems[k]) for k in range(PRIME)]
    late = [stage_idx(k, isem) for k in range(PRIME, K)]

    pos_copy = pltpu.async_copy(pos_hbm.at[pl.ds(s0, S_PER_W)], pos_v, psem)

    def gather_block(k):
        return pltpu.async_copy(
            emb_hbm.at[idx_v.at[k]], bufs[k % RING], gsems[k % RING])

    gathers = [None] * RING
    outs = [None] * RING
    for k in range(PRIME):
        early[k].wait()
        gathers[k] = gather_block(k)

    for c in late:
        c.wait()
    pos_copy.wait()

    for k in range(K):
        sl = k % RING
        kp = k + PRIME
        if kp < K:
            ps = kp % RING
            if outs[ps] is not None:
                outs[ps].wait()
                outs[ps] = None
            gathers[ps] = gather_block(kp)

        cur = bufs[sl]
        gathers[sl].wait()

        def add_pos(r, carry, cur=cur, base=(k % SPB) * SUB):
            for cc in range(D_MODEL // LANES):
                c = cc * LANES
                cur[r, pl.ds(c, LANES)] = (
                    cur[r, pl.ds(c, LANES)] + pos_v[base + r, pl.ds(c, LANES)])
            return carry

        lax.fori_loop(0, SUB, add_pos, 0)

        outs[sl] = pltpu.async_copy(
            cur, out_hbm.at[k // SPB, pl.ds(s0 + (k % SPB) * SUB, SUB)],
            osems[sl])

    for sl in range(RING):
        if outs[sl] is not None:
            outs[sl].wait()


def kernel(x, emb, pos_emb):
    return _emb_kernel(x.astype(jnp.int32), emb, pos_emb)
